# baseline (speedup 1.0000x reference)
import functools

import jax
import jax.numpy as jnp
from jax import lax
from jax.experimental import pallas as pl
from jax.experimental.pallas import tpu as pltpu
from jax.experimental.pallas import tpu_sc as plsc

_NUM_CORES = 2
_NUM_SUBCORES = 16
_NUM_WORKERS = _NUM_CORES * _NUM_SUBCORES

_CHUNK = 640


@functools.cache
def _build_rowmajor(d: int, vocab: int):
    n_blocks = (vocab + 127) // 128

    def body(t_ref, o_ref):
        tt = t_ref[...].T.reshape(d, 2, d)
        o_ref[...] = jnp.concatenate([tt[:, 0, :], tt[:, 1, :]], axis=1)

    return pl.pallas_call(
        body,
        grid=(n_blocks,),
        in_specs=[pl.BlockSpec((d, 128), lambda i: (0, i))],
        out_specs=pl.BlockSpec((d, 2 * d), lambda i: (i, 0)),
        out_shape=jax.ShapeDtypeStruct((vocab // 2, 2 * d), jnp.float32),
        compiler_params=pltpu.CompilerParams(
            dimension_semantics=("arbitrary",),
        ),
    )


@functools.cache
def _build_gather(n_total: int, vocab: int, d: int, dp: int):
    assert n_total % _NUM_WORKERS == 0
    n_per_w = n_total // _NUM_WORKERS
    chunk = min(_CHUNK, n_per_w)
    assert n_per_w % chunk == 0
    n_chunks = n_per_w // chunk
    assert n_chunks % 2 == 0 and n_chunks >= 2

    mesh = plsc.VectorSubcoreMesh(core_axis_name="c", subcore_axis_name="s")

    @functools.partial(
        pl.kernel,
        out_type=jax.ShapeDtypeStruct((n_total, dp), jnp.float32),
        mesh=mesh,
        scratch_types=[
            pltpu.VMEM((n_per_w,), jnp.int32),
            pltpu.VMEM((chunk, d), jnp.float32),
            pltpu.VMEM((chunk, d), jnp.float32),
            pltpu.SemaphoreType.DMA,
            pltpu.SemaphoreType.DMA,
            pltpu.SemaphoreType.DMA,
            pltpu.SemaphoreType.DMA,
        ],
        compiler_params=pltpu.CompilerParams(use_tc_tiling_on_sc=False),
    )
    def gather_kernel(idx_hbm, table_hbm, out_hbm, idx_v, rows0, rows1,
                      g0, g1, w0, w1):
        wid = lax.axis_index("s") * _NUM_CORES + lax.axis_index("c")
        base_w = wid * n_per_w
        rows = (rows0, rows1)
        gsem = (g0, g1)
        wsem = (w0, w1)

        pltpu.sync_copy(idx_hbm.at[pl.ds(base_w, n_per_w)], idx_v)

        def start_gather(j, b):
            pltpu.async_copy(
                table_hbm.at[idx_v.at[pl.ds(j * chunk, chunk)]],
                rows[b], gsem[b])

        def wait_gather(b):
            pltpu.make_async_copy(
                table_hbm.at[pl.ds(0, chunk)], rows[b], gsem[b]).wait()

        def out_window(i):
            return out_hbm.at[pl.ds(base_w + i * chunk, chunk), pl.ds(0, d)]

        start_gather(0, 0)
        start_gather(1, 1)

        def group(g, carry):
            for b in range(2):
                i = g * 2 + b
                wait_gather(b)
                wb = pltpu.make_async_copy(rows[b], out_window(i), wsem[b])
                wb.start()
                j = i + 2

                @pl.when(j < n_chunks)
                def _():
                    wb.wait()
                    start_gather(j, b)

            return carry

        lax.fori_loop(0, n_chunks // 2, group, 0, unroll=False)

        for b in range(2):
            i = n_chunks - 2 + b
            pltpu.make_async_copy(rows[b], out_window(i), wsem[b]).wait()

    return gather_kernel


def kernel(x, table):
    b, h = x.shape
    vocab, d = table.shape
    dp = 128
    table_rows = _build_rowmajor(d, vocab)(table.T)
    table_lin = table_rows.reshape(vocab, d)
    idx = x.reshape(b * h).astype(jnp.int32)
    out_p = _build_gather(b * h, vocab, d, dp)(idx, table_lin)
    return out_p[:, :d].reshape(b, h, d)


# speedup vs baseline: 4.1252x; 4.1252x over previous
import functools

import jax
import jax.numpy as jnp
from jax import lax
from jax.experimental import pallas as pl
from jax.experimental.pallas import tpu as pltpu
from jax.experimental.pallas import tpu_sc as plsc

_NUM_CORES = 2
_NUM_SUBCORES = 16
_NUM_WORKERS = _NUM_CORES * _NUM_SUBCORES

_CHUNK = 640


@functools.cache
def _build_rowmajor(d: int, vocab: int):
    fat = 8
    n_blocks = (vocab + 128 * fat - 1) // (128 * fat)

    def body(t_ref, o_ref):
        tt = t_ref[...].T.reshape(fat * d, 2, d)
        o_ref[...] = jnp.concatenate([tt[:, 0, :], tt[:, 1, :]], axis=1)

    return pl.pallas_call(
        body,
        grid=(n_blocks,),
        in_specs=[pl.BlockSpec((d, 128 * fat), lambda i: (0, i))],
        out_specs=pl.BlockSpec((fat * d, 2 * d), lambda i: (i, 0)),
        out_shape=jax.ShapeDtypeStruct((vocab // 2, 2 * d), jnp.float32),
        compiler_params=pltpu.CompilerParams(
            dimension_semantics=("arbitrary",),
        ),
    )


@functools.cache
def _build_gather(n_total: int, vocab: int, d: int, dp: int):
    assert n_total % _NUM_WORKERS == 0
    n_per_w = n_total // _NUM_WORKERS
    chunk = min(_CHUNK, n_per_w)
    assert n_per_w % chunk == 0
    n_chunks = n_per_w // chunk
    assert n_chunks % 2 == 0 and n_chunks >= 2

    mesh = plsc.VectorSubcoreMesh(core_axis_name="c", subcore_axis_name="s")

    @functools.partial(
        pl.kernel,
        out_type=jax.ShapeDtypeStruct((n_total, dp), jnp.float32),
        mesh=mesh,
        scratch_types=[
            pltpu.VMEM((n_per_w,), jnp.int32),
            pltpu.VMEM((chunk, d), jnp.float32),
            pltpu.VMEM((chunk, d), jnp.float32),
            pltpu.SemaphoreType.DMA,
            pltpu.SemaphoreType.DMA,
            pltpu.SemaphoreType.DMA,
            pltpu.SemaphoreType.DMA,
        ],
        compiler_params=pltpu.CompilerParams(use_tc_tiling_on_sc=False),
    )
    def gather_kernel(idx_hbm, table_hbm, out_hbm, idx_v, rows0, rows1,
                      g0, g1, w0, w1):
        wid = lax.axis_index("s") * _NUM_CORES + lax.axis_index("c")
        base_w = wid * n_per_w
        rows = (rows0, rows1)
        gsem = (g0, g1)
        wsem = (w0, w1)

        pltpu.sync_copy(idx_hbm.at[pl.ds(base_w, n_per_w)], idx_v)

        def start_gather(j, b):
            pltpu.async_copy(
                table_hbm.at[idx_v.at[pl.ds(j * chunk, chunk)]],
                rows[b], gsem[b])

        def wait_gather(b):
            pltpu.make_async_copy(
                table_hbm.at[pl.ds(0, chunk)], rows[b], gsem[b]).wait()

        def out_window(i):
            return out_hbm.at[pl.ds(base_w + i * chunk, chunk), pl.ds(0, d)]

        start_gather(0, 0)
        start_gather(1, 1)

        def group(g, carry):
            for b in range(2):
                i = g * 2 + b
                wait_gather(b)
                wb = pltpu.make_async_copy(rows[b], out_window(i), wsem[b])
                wb.start()
                j = i + 2

                @pl.when(j < n_chunks)
                def _():
                    wb.wait()
                    start_gather(j, b)

            return carry

        lax.fori_loop(0, n_chunks // 2, group, 0, unroll=False)

        for b in range(2):
            i = n_chunks - 2 + b
            pltpu.make_async_copy(rows[b], out_window(i), wsem[b]).wait()

    return gather_kernel


def kernel(x, table):
    b, h = x.shape
    vocab, d = table.shape
    dp = 128
    table_rows = _build_rowmajor(d, vocab)(table.T)
    table_lin = table_rows.reshape(vocab, d)
    idx = x.reshape(b * h).astype(jnp.int32)
    out_p = _build_gather(b * h, vocab, d, dp)(idx, table_lin)
    return out_p[:, :d].reshape(b, h, d)
